# baseline (speedup 1.0000x reference)
import jax
import jax.numpy as jnp
from jax import lax
from jax.experimental import pallas as pl
from jax.experimental.pallas import tpu as pltpu

E = 8
K = 2
D = 1024
F = 2048
T = 512
KT = K * T
TS = 256
NTILES = 11
NS = NTILES * TS
FC = 512
NF = F // FC


def _router_body(x_ref, wg_ref, tok_ref, pb_ref, te_ref, va_ref):
    x = x_ref[...]
    wg = wg_ref[...]
    scores = jnp.dot(x, wg, preferred_element_type=jnp.float32,
                     precision=lax.Precision.HIGHEST)
    cols = lax.broadcasted_iota(jnp.int32, (T, E), 1)
    m1 = jnp.max(scores, axis=1, keepdims=True)
    e1 = jnp.min(jnp.where(scores == m1, cols, E), axis=1)
    neg = jnp.float32(-jnp.inf)
    sc2 = jnp.where(cols == e1[:, None], neg, scores)
    m2 = jnp.max(sc2, axis=1, keepdims=True)
    e2 = jnp.min(jnp.where(sc2 == m2, cols, E), axis=1)
    z = jnp.exp(m2[:, 0] - m1[:, 0])
    p1 = 1.0 / (1.0 + z)
    p2 = 1.0 - p1
    a = jnp.concatenate([e1, e2])
    p = jnp.concatenate([p1, p2])

    ecols = lax.broadcasted_iota(jnp.int32, (KT, E), 1)
    M = (a[:, None] == ecols).astype(jnp.float32)
    ri = lax.broadcasted_iota(jnp.int32, (KT, KT), 0)
    ci = lax.broadcasted_iota(jnp.int32, (KT, KT), 1)
    L = (ri > ci).astype(jnp.float32)
    R = jnp.dot(L, M, preferred_element_type=jnp.float32,
                precision=lax.Precision.HIGHEST)
    rank = jnp.sum(M * R, axis=1)

    c = jnp.sum(M, axis=0)
    nt = jnp.ceil(c * (1.0 / TS))
    ei = lax.broadcasted_iota(jnp.int32, (E, E), 0)
    ej = lax.broadcasted_iota(jnp.int32, (E, E), 1)
    po = jnp.sum(jnp.where(ej < ei, (nt * TS)[None, :], 0.0), axis=1)
    po_end = po + nt * TS
    total = jnp.sum(nt) * TS

    po_a = jnp.sum(M * po[None, :], axis=1)
    pos = (po_a + rank).astype(jnp.int32)

    jj = lax.broadcasted_iota(jnp.int32, (KT, NS), 1)
    A = (pos[:, None] == jj).astype(jnp.float32)
    toki = lax.iota(jnp.int32, T).astype(jnp.float32)
    tokf = jnp.concatenate([toki, toki])
    tok_ref[...] = jnp.dot(tokf[None, :], A, preferred_element_type=jnp.float32,
                           precision=lax.Precision.HIGHEST).astype(jnp.int32)
    pb_ref[...] = jnp.dot(p[None, :], A, preferred_element_type=jnp.float32,
                          precision=lax.Precision.HIGHEST)

    starts = (lax.broadcasted_iota(jnp.int32, (1, 128), 1) * TS
              ).astype(jnp.float32)
    cmp = (po_end[:, None] <= starts).astype(jnp.int32)
    te_raw = jnp.sum(cmp, axis=0, keepdims=True)
    eid = lax.broadcasted_iota(jnp.int32, (1, E), 1)
    last_e = jnp.max(jnp.where(c[None, :] > 0, eid, -1))
    valid = starts < total
    te_ref[...] = jnp.where(valid, jnp.clip(te_raw, 0, E - 1), last_e)
    va_ref[...] = valid.astype(jnp.int32)


def _router(xf, Wg):
    return pl.pallas_call(
        _router_body,
        out_shape=(
            jax.ShapeDtypeStruct((1, NS), jnp.int32),
            jax.ShapeDtypeStruct((1, NS), jnp.float32),
            jax.ShapeDtypeStruct((1, 128), jnp.int32),
            jax.ShapeDtypeStruct((1, 128), jnp.int32),
        ),
    )(xf, Wg)


def _mega_body(te_ref, va_ref, x_ref, w1_ref, w2_ref, w3_ref, tok_ref, pb_ref,
               y_ref, xs_sc, acc_sc):
    i = pl.program_id(0)
    f = pl.program_id(1)

    @pl.when(jnp.logical_and(i == 0, f == 0))
    def _():
        y_ref[...] = jnp.zeros_like(y_ref)

    @pl.when(va_ref[i] != 0)
    def _():
        tok = tok_ref[0, 0]

        @pl.when(f == 0)
        def _():
            tcols = lax.broadcasted_iota(jnp.int32, (TS, T), 1)
            oh = (tok[:, None] == tcols).astype(jnp.float32)
            xs_sc[...] = jnp.dot(oh, x_ref[...],
                                 preferred_element_type=jnp.float32)

        xt = xs_sc[...]
        w1 = w1_ref[0]
        w2 = w2_ref[0]
        w3 = w3_ref[0]
        g = jnp.dot(xt, w1, preferred_element_type=jnp.float32)
        u = jnp.dot(xt, w2, preferred_element_type=jnp.float32)
        contrib = jnp.dot((g * (1.0 / (1.0 + jnp.exp(-g)))) * u, w3,
                          preferred_element_type=jnp.float32)

        @pl.when(f == 0)
        def _():
            acc_sc[...] = contrib

        @pl.when(f != 0)
        def _():
            acc_sc[...] += contrib

        @pl.when(f == NF - 1)
        def _():
            trows = lax.broadcasted_iota(jnp.int32, (T, TS), 0)
            c2 = jnp.where(tok[None, :] == trows, pb_ref[0, 0][None, :], 0.0)
            y_ref[...] += jnp.dot(c2, acc_sc[...],
                                  preferred_element_type=jnp.float32)


def _mega(xf, W1, W2, W3, tok_slot, pb_slot, te, valid):
    def fe(i, f, te_r, va_r):
        return te_r[i]

    def fw(i, f, te_r, va_r):
        return jnp.where(va_r[i] != 0, f, NF - 1)

    grid_spec = pltpu.PrefetchScalarGridSpec(
        num_scalar_prefetch=2,
        grid=(NTILES, NF),
        in_specs=[
            pl.BlockSpec((T, D), lambda i, f, te_r, va_r: (0, 0)),
            pl.BlockSpec((1, D, FC), lambda i, f, te_r, va_r:
                         (fe(i, f, te_r, va_r), 0, fw(i, f, te_r, va_r))),
            pl.BlockSpec((1, D, FC), lambda i, f, te_r, va_r:
                         (fe(i, f, te_r, va_r), 0, fw(i, f, te_r, va_r))),
            pl.BlockSpec((1, FC, D), lambda i, f, te_r, va_r:
                         (fe(i, f, te_r, va_r), fw(i, f, te_r, va_r), 0)),
            pl.BlockSpec((1, 1, TS), lambda i, f, te_r, va_r: (i, 0, 0)),
            pl.BlockSpec((1, 1, TS), lambda i, f, te_r, va_r: (i, 0, 0)),
        ],
        out_specs=pl.BlockSpec((T, D), lambda i, f, te_r, va_r: (0, 0)),
        scratch_shapes=[
            pltpu.VMEM((TS, D), jnp.float32),
            pltpu.VMEM((TS, D), jnp.float32),
        ],
    )
    return pl.pallas_call(
        _mega_body,
        grid_spec=grid_spec,
        out_shape=jax.ShapeDtypeStruct((T, D), jnp.float32),
    )(te, valid, xf, W1, W2, W3, tok_slot, pb_slot)


def kernel(x, Wg, W1, W2, W3):
    b, s, d = x.shape
    xf = x.reshape(b * s, d)
    tok_slot, pb_slot, te, valid = _router(xf, Wg)
    y = _mega(
        xf, W1, W2, W3,
        tok_slot.reshape(NTILES, 1, TS),
        pb_slot.reshape(NTILES, 1, TS),
        te.reshape(128)[:NTILES],
        valid.reshape(128)[:NTILES],
    )
    return y.reshape(b, s, d)


# speedup vs baseline: 1.2266x; 1.2266x over previous
import jax
import jax.numpy as jnp
from jax import lax
from jax.experimental import pallas as pl
from jax.experimental.pallas import tpu as pltpu

E = 8
K = 2
D = 1024
F = 2048
T = 512
KT = K * T
TS = 256
NTILES = 11
NS = NTILES * TS
FC = 1024
NF = F // FC


def _router_body(x_ref, wg_ref, tok_ref, pb_ref, te_ref, va_ref):
    x = x_ref[...]
    wg = wg_ref[...]
    scores = jnp.dot(x, wg, preferred_element_type=jnp.float32,
                     precision=lax.Precision.HIGHEST)
    cols = lax.broadcasted_iota(jnp.int32, (T, E), 1)
    m1 = jnp.max(scores, axis=1, keepdims=True)
    e1 = jnp.min(jnp.where(scores == m1, cols, E), axis=1)
    neg = jnp.float32(-jnp.inf)
    sc2 = jnp.where(cols == e1[:, None], neg, scores)
    m2 = jnp.max(sc2, axis=1, keepdims=True)
    e2 = jnp.min(jnp.where(sc2 == m2, cols, E), axis=1)
    z = jnp.exp(m2[:, 0] - m1[:, 0])
    p1 = 1.0 / (1.0 + z)
    p2 = 1.0 - p1
    a = jnp.concatenate([e1, e2])
    p = jnp.concatenate([p1, p2])

    ecols = lax.broadcasted_iota(jnp.int32, (KT, E), 1)
    M = (a[:, None] == ecols).astype(jnp.float32)
    ri = lax.broadcasted_iota(jnp.int32, (KT, KT), 0)
    ci = lax.broadcasted_iota(jnp.int32, (KT, KT), 1)
    L = (ri > ci).astype(jnp.float32)
    R = jnp.dot(L, M, preferred_element_type=jnp.float32)
    rank = jnp.sum(M * R, axis=1)

    c = jnp.sum(M, axis=0)
    nt = jnp.ceil(c * (1.0 / TS))
    ei = lax.broadcasted_iota(jnp.int32, (E, E), 0)
    ej = lax.broadcasted_iota(jnp.int32, (E, E), 1)
    po = jnp.sum(jnp.where(ej < ei, (nt * TS)[None, :], 0.0), axis=1)
    po_end = po + nt * TS
    total = jnp.sum(nt) * TS

    po_a = jnp.sum(M * po[None, :], axis=1)
    pos = (po_a + rank).astype(jnp.int32)

    tile_i = jnp.right_shift(pos, 8)
    r_i = jnp.bitwise_and(pos, 255)
    trow = lax.broadcasted_iota(jnp.int32, (16, KT), 0)
    a1t = (tile_i[None, :] == trow).astype(jnp.float32)
    rcol = lax.broadcasted_iota(jnp.int32, (KT, TS), 1)
    a2 = (r_i[:, None] == rcol).astype(jnp.float32)
    toki = lax.iota(jnp.int32, T)
    tok = jnp.concatenate([toki, toki])
    tokl = jnp.bitwise_and(tok, 255).astype(jnp.float32)
    tokh = jnp.right_shift(tok, 8).astype(jnp.float32)
    outl = jnp.dot(a1t, a2 * tokl[:, None], preferred_element_type=jnp.float32)
    outh = jnp.dot(a1t, a2 * tokh[:, None], preferred_element_type=jnp.float32)
    tok_ref[...] = (outl + 256.0 * outh).astype(jnp.int32)
    pb_ref[...] = jnp.dot(a1t, a2 * p[:, None],
                          preferred_element_type=jnp.float32,
                          precision=lax.Precision.HIGHEST)

    starts = (lax.broadcasted_iota(jnp.int32, (1, 128), 1) * TS
              ).astype(jnp.float32)
    cmp = (po_end[:, None] <= starts).astype(jnp.int32)
    te_raw = jnp.sum(cmp, axis=0, keepdims=True)
    eid = lax.broadcasted_iota(jnp.int32, (1, E), 1)
    last_e = jnp.max(jnp.where(c[None, :] > 0, eid, -1))
    valid = starts < total
    te_ref[...] = jnp.where(valid, jnp.clip(te_raw, 0, E - 1), last_e)
    va_ref[...] = valid.astype(jnp.int32)


def _router(xf, Wg):
    return pl.pallas_call(
        _router_body,
        out_shape=(
            jax.ShapeDtypeStruct((16, TS), jnp.int32),
            jax.ShapeDtypeStruct((16, TS), jnp.float32),
            jax.ShapeDtypeStruct((1, 128), jnp.int32),
            jax.ShapeDtypeStruct((1, 128), jnp.int32),
        ),
    )(xf, Wg)


def _mega_body(te_ref, va_ref, x_ref, w1_ref, w2_ref, w3_ref, tok_ref, pb_ref,
               y_ref, xs_sc, acc_sc):
    i = pl.program_id(0)
    f = pl.program_id(1)

    @pl.when(jnp.logical_and(i == 0, f == 0))
    def _():
        y_ref[...] = jnp.zeros_like(y_ref)

    @pl.when(va_ref[i] != 0)
    def _():
        tok = tok_ref[0, 0]

        @pl.when(f == 0)
        def _():
            tcols = lax.broadcasted_iota(jnp.int32, (TS, T), 1)
            oh = (tok[:, None] == tcols).astype(jnp.float32)
            xs_sc[...] = jnp.dot(oh, x_ref[...],
                                 preferred_element_type=jnp.float32)

        xt = xs_sc[...]
        w1 = w1_ref[0]
        w2 = w2_ref[0]
        w3 = w3_ref[0]
        g = jnp.dot(xt, w1, preferred_element_type=jnp.float32)
        u = jnp.dot(xt, w2, preferred_element_type=jnp.float32)
        contrib = jnp.dot((g * (1.0 / (1.0 + jnp.exp(-g)))) * u, w3,
                          preferred_element_type=jnp.float32)

        @pl.when(f == 0)
        def _():
            acc_sc[...] = contrib

        @pl.when(f != 0)
        def _():
            acc_sc[...] += contrib

        @pl.when(f == NF - 1)
        def _():
            trows = lax.broadcasted_iota(jnp.int32, (T, TS), 0)
            c2 = jnp.where(tok[None, :] == trows, pb_ref[0, 0][None, :], 0.0)
            y_ref[...] += jnp.dot(c2, acc_sc[...],
                                  preferred_element_type=jnp.float32)


def _mega(xf, W1, W2, W3, tok_slot, pb_slot, te, valid):
    def fe(i, f, te_r, va_r):
        return te_r[i]

    def fw(i, f, te_r, va_r):
        return jnp.where(va_r[i] != 0, f, NF - 1)

    grid_spec = pltpu.PrefetchScalarGridSpec(
        num_scalar_prefetch=2,
        grid=(NTILES, NF),
        in_specs=[
            pl.BlockSpec((T, D), lambda i, f, te_r, va_r: (0, 0)),
            pl.BlockSpec((1, D, FC), lambda i, f, te_r, va_r:
                         (fe(i, f, te_r, va_r), 0, fw(i, f, te_r, va_r))),
            pl.BlockSpec((1, D, FC), lambda i, f, te_r, va_r:
                         (fe(i, f, te_r, va_r), 0, fw(i, f, te_r, va_r))),
            pl.BlockSpec((1, FC, D), lambda i, f, te_r, va_r:
                         (fe(i, f, te_r, va_r), fw(i, f, te_r, va_r), 0)),
            pl.BlockSpec((1, 1, TS), lambda i, f, te_r, va_r: (i, 0, 0)),
            pl.BlockSpec((1, 1, TS), lambda i, f, te_r, va_r: (i, 0, 0)),
        ],
        out_specs=pl.BlockSpec((T, D), lambda i, f, te_r, va_r: (0, 0)),
        scratch_shapes=[
            pltpu.VMEM((TS, D), jnp.float32),
            pltpu.VMEM((TS, D), jnp.float32),
        ],
    )
    return pl.pallas_call(
        _mega_body,
        grid_spec=grid_spec,
        out_shape=jax.ShapeDtypeStruct((T, D), jnp.float32),
    )(te, valid, xf, W1, W2, W3, tok_slot, pb_slot)


def kernel(x, Wg, W1, W2, W3):
    b, s, d = x.shape
    xf = x.reshape(b * s, d)
    tok_slot, pb_slot, te, valid = _router(xf, Wg)
    y = _mega(
        xf, W1, W2, W3,
        tok_slot[:NTILES].reshape(NTILES, 1, TS),
        pb_slot[:NTILES].reshape(NTILES, 1, TS),
        te.reshape(128)[:NTILES],
        valid.reshape(128)[:NTILES],
    )
    return y.reshape(b, s, d)
